# baseline (speedup 1.0000x reference)
import functools

import jax
import jax.numpy as jnp
from jax import lax
from jax.experimental import pallas as pl
from jax.experimental.pallas import tpu as pltpu
from jax.experimental.pallas import tpu_sc as plsc

N = 10000
E = 320000
D = 128

NC = 2
NS = 16
NW = NC * NS
EPW = E // NW
DH = D // 2
EPT = E // NS
CH = 200
NCHUNK = EPT // CH
K = 5
NG = NCHUNK // K
CD = 80
ND = EPW // CD
KD = 5
NGD = ND // KD
RA = 624
RB = N - (NS - 1) * RA

_BN_INV = 0.9999950000374997


@functools.lru_cache(maxsize=None)
def _sc_kernels():
    mesh = plsc.VectorSubcoreMesh(core_axis_name="c", subcore_axis_name="s")

    def _init_and_readback(sid, cid, zeros_hbm, sh, out_hbm, phase):
        start = pl.multiple_of(sid * RA, 8)

        @pl.when(sid < NS - 1)
        def _():
            if phase == 0:
                pltpu.sync_copy(zeros_hbm.at[pl.ds(0, RA)],
                                sh.at[pl.ds(start, RA)])
            else:
                pltpu.sync_copy(sh.at[pl.ds(start, RA)],
                                out_hbm.at[cid, pl.ds(start, RA)])

        @pl.when(sid == NS - 1)
        def _():
            if phase == 0:
                pltpu.sync_copy(zeros_hbm,
                                sh.at[pl.ds((NS - 1) * RA, RB)])
            else:
                pltpu.sync_copy(sh.at[pl.ds((NS - 1) * RA, RB)],
                                out_hbm.at[cid, pl.ds((NS - 1) * RA, RB)])

    @functools.partial(
        pl.kernel,
        out_type=jax.ShapeDtypeStruct((NC, N, 16), jnp.float32),
        mesh=mesh,
        scratch_types=[
            [pltpu.VMEM((CD,), jnp.int32)] * KD,
            pltpu.VMEM((CD, 16), jnp.float32),
            pltpu.VMEM_SHARED((N, 16), jnp.float32),
            [pltpu.SemaphoreType.DMA] * KD,
            [pltpu.SemaphoreType.DMA] * KD,
        ],
        compiler_params=pltpu.CompilerParams(use_tc_tiling_on_sc=False),
    )
    def sc_degree(col_hbm, ones_hbm, zeros_hbm, out_hbm, cidx_vs, ones_v,
                  deg_sh, isems, ssems):
        cid = lax.axis_index("c")
        sid = lax.axis_index("s")
        wid = sid * NC + cid
        _init_and_readback(sid, cid, zeros_hbm, deg_sh, out_hbm, 0)
        pltpu.sync_copy(ones_hbm, ones_v)

        def isrc(i):
            return col_hbm.at[pl.ds(pl.multiple_of(wid * EPW + i * CD, 8), CD)]

        def iload(i, b):
            pltpu.async_copy(isrc(i), cidx_vs[b], isems[b])

        def iwait(i, b):
            pltpu.make_async_copy(isrc(i), cidx_vs[b], isems[b]).wait()

        def scatter(b):
            pltpu.async_copy(ones_v, deg_sh.at[cidx_vs[b]], ssems[b], add=True)

        def scatter_wait(b):
            pltpu.make_async_copy(ones_v, deg_sh.at[cidx_vs[b]],
                                  ssems[b]).wait()

        for b in range(KD):
            iload(b, b)
        plsc.subcore_barrier()

        def group(j, carry):
            for b in range(KD):
                i = j * KD + b
                iwait(i, b)
                scatter(b)

            @pl.when(j < NGD - 1)
            def _():
                for b in range(KD):
                    scatter_wait(b)
                    iload((j + 1) * KD + b, b)

            return carry

        lax.fori_loop(0, NGD, group, 0)
        for b in range(KD):
            scatter_wait(b)
        plsc.subcore_barrier()
        _init_and_readback(sid, cid, zeros_hbm, deg_sh, out_hbm, 1)

    @functools.partial(
        pl.kernel,
        out_type=jax.ShapeDtypeStruct((NC, N, DH), jnp.float32),
        mesh=mesh,
        scratch_types=[
            [pltpu.VMEM((K * CH,), jnp.int32)] * 2,
            [pltpu.VMEM((CH,), jnp.int32)] * (2 * K),
            pltpu.VMEM((K * CH, DH), jnp.float32),
            pltpu.VMEM_SHARED((N, DH), jnp.float32),
            [pltpu.SemaphoreType.DMA] * K,
            [pltpu.SemaphoreType.DMA] * K,
            [pltpu.SemaphoreType.DMA] * 2,
            [pltpu.SemaphoreType.DMA] * 2,
        ],
        compiler_params=pltpu.CompilerParams(use_tc_tiling_on_sc=False),
    )
    def sc_spmm(g3_hbm, row_hbm, col_hbm, zeros_hbm, out_hbm,
                ridx_vs, cidx_vs, rows_v, agg_sh, gsems, ssems, irsems, icsems):
        cid = lax.axis_index("c")
        sid = lax.axis_index("s")
        _init_and_readback(sid, cid, zeros_hbm, agg_sh, out_hbm, 0)

        SLAB = K * CH

        def slab_refs(j, d):
            base = pl.multiple_of(sid * EPT + j * SLAB, 8)
            refs = [(row_hbm.at[pl.ds(base, SLAB)], ridx_vs[d], irsems[d])]
            for b in range(K):
                cb = pl.multiple_of(base + b * CH, 8)
                refs.append((col_hbm.at[pl.ds(cb, CH)], cidx_vs[d * K + b],
                             icsems[d]))
            return refs

        def slab_load(j, d):
            for s, dst, sem in slab_refs(j, d):
                pltpu.async_copy(s, dst, sem)

        def slab_wait(j, d):
            for s, dst, sem in slab_refs(j, d):
                pltpu.make_async_copy(s, dst, sem).wait()

        def gsrc(d, b):
            return g3_hbm.at[cid].at[ridx_vs[d].at[pl.ds(b * CH, CH)]]

        def rbuf(b):
            return rows_v.at[pl.ds(b * CH, CH)]

        def gather(d, b):
            pltpu.async_copy(gsrc(d, b), rbuf(b), gsems[b])

        def gather_wait(d, b):
            pltpu.make_async_copy(gsrc(d, b), rbuf(b), gsems[b]).wait()

        def scatter(d, b):
            pltpu.async_copy(rbuf(b), agg_sh.at[cidx_vs[d * K + b]],
                             ssems[b], add=True)

        def scatter_wait(d, b):
            pltpu.make_async_copy(rbuf(b), agg_sh.at[cidx_vs[d * K + b]],
                                  ssems[b]).wait()

        slab_load(0, 0)
        slab_wait(0, 0)
        plsc.subcore_barrier()
        for b in range(K):
            gather(0, b)

        def one_group(j, d):
            nd = 1 - d

            @pl.when(j < NG - 1)
            def _():
                slab_load(j + 1, nd)

            for b in range(K):
                gather_wait(d, b)
                scatter(d, b)

            @pl.when(j < NG - 1)
            def _():
                slab_wait(j + 1, nd)
                for b in range(K):
                    scatter_wait(d, b)
                    gather(nd, b)

        def pair(t, carry):
            one_group(2 * t, 0)
            one_group(2 * t + 1, 1)
            return carry

        lax.fori_loop(0, NG // 2, pair, 0)
        for b in range(K):
            scatter_wait((NG - 1) % 2, b)
        plsc.subcore_barrier()
        _init_and_readback(sid, cid, zeros_hbm, agg_sh, out_hbm, 1)

    return sc_degree, sc_spmm


R = 1000
G = N // R


def _dot_t(a, w):
    return lax.dot_general(a, w, (((1,), (1,)), ((), ())),
                           preferred_element_type=jnp.float32)


def _ln(h, g, b):
    mu = jnp.mean(h, axis=-1, keepdims=True)
    var = jnp.mean((h - mu) * (h - mu), axis=-1, keepdims=True)
    return (h - mu) * lax.rsqrt(var + 1e-5) * g + b


def _ka_body(x_ref, tfw, tfb, ln0g, ln0b, wq, wqb, wk, wkb, wv, wvb,
             gfw, gfb, bn0g, bn0b,
             layer0_ref, qs_ref, g0_ref, kvs_ref, vsum_ref, ksum_ref):
    i = pl.program_id(0)
    xb = x_ref[...]
    h = _dot_t(xb, tfw[...]) + tfb[...]
    h = jnp.maximum(_ln(h, ln0g[...], ln0b[...]), 0.0)
    layer0_ref[...] = h
    qs = _dot_t(h, wq[...]) + wqb[...]
    kk = _dot_t(h, wk[...]) + wkb[...]
    vs = _dot_t(h, wv[...]) + wvb[...]
    qs = qs * lax.rsqrt(jnp.sum(qs * qs, axis=-1, keepdims=True))
    kk = kk * lax.rsqrt(jnp.sum(kk * kk, axis=-1, keepdims=True))
    qs_ref[...] = qs
    kv = lax.dot_general(kk, vs, (((0,), (0,)), ((), ())),
                         preferred_element_type=jnp.float32)
    vsum = jnp.sum(vs, axis=0, keepdims=True)
    ksum = jnp.sum(kk, axis=0, keepdims=True)

    @pl.when(i == 0)
    def _():
        kvs_ref[...] = kv
        vsum_ref[...] = vsum
        ksum_ref[...] = ksum

    @pl.when(i > 0)
    def _():
        kvs_ref[...] += kv
        vsum_ref[...] += vsum
        ksum_ref[...] += ksum

    g0 = _dot_t(xb, gfw[...]) + gfb[...]
    g0 = jnp.maximum(g0 * _BN_INV * bn0g[...] + bn0b[...], 0.0)
    g0_ref[...] = g0


def _scale_from_deg(deg_ref):
    degb = deg_ref[0, :, 0:1] + deg_ref[1, :, 0:1]
    return jnp.where(degb > 0.0, lax.rsqrt(degb), 0.0)


def _split_store(out_ref, t):
    out_ref[0] = t[:, :DH]
    out_ref[1] = t[:, DH:]


def _ks_body(deg_ref, g0_ref, out_ref):
    _split_store(out_ref, _scale_from_deg(deg_ref) * g0_ref[...])


def _kb_body(p_ref, deg_ref, g0_ref, w, b, bng, bnb, out_ref):
    s = _scale_from_deg(deg_ref)
    agg = jnp.concatenate([p_ref[0], p_ref[1]], axis=-1) * s
    t = _dot_t(agg, w[...]) + b[...]
    t = jnp.maximum(t * _BN_INV * bng[...] + bnb[...], 0.0) + g0_ref[...]
    _split_store(out_ref, s * t)


def _kd_body(p_ref, deg_ref, g0_ref, qs_ref, layer0_ref,
             kvs_ref, vsum_ref, ksum_ref,
             w, b, bng, bnb, ln1g, ln1b, ow, ob, out_ref):
    s = _scale_from_deg(deg_ref)
    agg = jnp.concatenate([p_ref[0], p_ref[1]], axis=-1) * s
    t = _dot_t(agg, w[...]) + b[...]
    x2 = jnp.maximum(t * _BN_INV * bng[...] + bnb[...], 0.0) + g0_ref[...]

    qs = qs_ref[...]
    num = jnp.dot(qs, kvs_ref[...], preferred_element_type=jnp.float32)
    num = num + jnp.float32(N) * vsum_ref[...]
    den = jnp.sum(qs * ksum_ref[...], axis=-1, keepdims=True)
    den = den + jnp.float32(2 * N)
    h = (num / den + layer0_ref[...]) * 0.5
    x1 = jnp.maximum(_ln(h, ln1g[...], ln1b[...]), 0.0)

    out_ref[...] = _dot_t(0.8 * x2 + 0.2 * x1, ow[...]) + ob[...]


_WSPEC = pl.BlockSpec((D, D), lambda i: (0, 0))
_BSPEC = pl.BlockSpec((1, D), lambda i: (0, 0))
_XSPEC = pl.BlockSpec((R, D), lambda i: (i, 0))
_DEGSPEC = pl.BlockSpec((NC, R, 16), lambda i: (0, i, 0))
_PSPEC = pl.BlockSpec((NC, R, DH), lambda i: (0, i, 0))

_f32 = jnp.float32


def _ka(x, tfw, tfb, ln0g, ln0b, wq, wqb, wk, wkb, wv, wvb, gfw, gfb, bn0g, bn0b):
    outs = (
        jax.ShapeDtypeStruct((N, D), _f32),
        jax.ShapeDtypeStruct((N, D), _f32),
        jax.ShapeDtypeStruct((N, D), _f32),
        jax.ShapeDtypeStruct((D, D), _f32),
        jax.ShapeDtypeStruct((1, D), _f32),
        jax.ShapeDtypeStruct((1, D), _f32),
    )
    in_specs = [_XSPEC, _WSPEC, _BSPEC, _BSPEC, _BSPEC,
                _WSPEC, _BSPEC, _WSPEC, _BSPEC, _WSPEC, _BSPEC,
                _WSPEC, _BSPEC, _BSPEC, _BSPEC]
    out_specs = (_XSPEC, _XSPEC, _XSPEC, _WSPEC, _BSPEC, _BSPEC)
    return pl.pallas_call(
        _ka_body, grid=(G,), in_specs=in_specs, out_specs=out_specs,
        out_shape=outs)(x, tfw, tfb, ln0g, ln0b, wq, wqb, wk, wkb, wv, wvb,
                        gfw, gfb, bn0g, bn0b)


def _ks(deg, g0):
    return pl.pallas_call(
        _ks_body, grid=(G,), in_specs=[_DEGSPEC, _XSPEC], out_specs=_PSPEC,
        out_shape=jax.ShapeDtypeStruct((NC, N, DH), _f32))(deg, g0)


def _kb(p, deg, g0, w, b, bng, bnb):
    return pl.pallas_call(
        _kb_body, grid=(G,),
        in_specs=[_PSPEC, _DEGSPEC, _XSPEC, _WSPEC, _BSPEC, _BSPEC, _BSPEC],
        out_specs=_PSPEC,
        out_shape=jax.ShapeDtypeStruct((NC, N, DH), _f32))(p, deg, g0, w, b,
                                                           bng, bnb)


def _kd(p, deg, g0, qs, layer0, kvs, vsum, ksum, w, b, bng, bnb,
        ln1g, ln1b, ow, ob):
    return pl.pallas_call(
        _kd_body, grid=(G,),
        in_specs=[_PSPEC, _DEGSPEC, _XSPEC, _XSPEC, _XSPEC,
                  _WSPEC, _BSPEC, _BSPEC,
                  _WSPEC, _BSPEC, _BSPEC, _BSPEC, _BSPEC, _BSPEC,
                  _WSPEC, _BSPEC],
        out_specs=_XSPEC,
        out_shape=jax.ShapeDtypeStruct((N, D), _f32))(
            p, deg, g0, qs, layer0, kvs, vsum, ksum,
            w, b, bng, bnb, ln1g, ln1b, ow, ob)


def kernel(x, edge_index, t_fc_w, t_fc_b, t_ln0_g, t_ln0_b, t_wq_w, t_wq_b,
           t_wk_w, t_wk_b, t_wv_w, t_wv_b, t_ln1_g, t_ln1_b, g_fc_w, g_fc_b,
           g_bn0_g, g_bn0_b, g_w1_w, g_w1_b, g_bn1_g, g_bn1_b, g_w2_w, g_w2_b,
           g_bn2_g, g_bn2_b, out_w, out_b):
    row = edge_index[0]
    col = edge_index[1]

    r2 = lambda v: v.reshape(1, D)

    ones16 = jnp.ones((CD, 16), _f32)
    zerosD = jnp.zeros((RB, DH), _f32)
    zeros16 = jnp.zeros((RB, 16), _f32)

    layer0, qs, g0, kvs, vsum, ksum = _ka(
        x, t_fc_w, r2(t_fc_b), r2(t_ln0_g), r2(t_ln0_b),
        t_wq_w, r2(t_wq_b), t_wk_w, r2(t_wk_b), t_wv_w, r2(t_wv_b),
        g_fc_w, r2(g_fc_b), r2(g_bn0_g), r2(g_bn0_b))

    sc_degree, sc_spmm = _sc_kernels()
    deg = sc_degree(col, ones16, zeros16)

    g1s = _ks(deg, g0)
    p1 = sc_spmm(g1s, row, col, zerosD)
    g2s = _kb(p1, deg, g0, g_w1_w, r2(g_w1_b), r2(g_bn1_g), r2(g_bn1_b))
    p2 = sc_spmm(g2s, row, col, zerosD)
    out = _kd(p2, deg, g0, qs, layer0, kvs, vsum, ksum,
              g_w2_w, r2(g_w2_b), r2(g_bn2_g), r2(g_bn2_b),
              r2(t_ln1_g), r2(t_ln1_b), out_w, r2(out_b))
    return out


# speedup vs baseline: 1.0045x; 1.0045x over previous
import functools

import jax
import jax.numpy as jnp
from jax import lax
from jax.experimental import pallas as pl
from jax.experimental.pallas import tpu as pltpu
from jax.experimental.pallas import tpu_sc as plsc

N = 10000
E = 320000
D = 128

NC = 2
NS = 16
NW = NC * NS
EPW = E // NW
DH = D // 2
EPT = E // NS
CH = 200
NCHUNK = EPT // CH
K = 5
NG = NCHUNK // K
CD = 80
ND = EPW // CD
KD = 5
NGD = ND // KD
RA = 624
RB = N - (NS - 1) * RA

_BN_INV = 0.9999950000374997


@functools.lru_cache(maxsize=None)
def _sc_kernels():
    mesh = plsc.VectorSubcoreMesh(core_axis_name="c", subcore_axis_name="s")

    def _init_and_readback(sid, cid, zeros_hbm, sh, out_hbm, phase):
        start = pl.multiple_of(sid * RA, 8)

        @pl.when(sid < NS - 1)
        def _():
            if phase == 0:
                pltpu.sync_copy(zeros_hbm.at[pl.ds(0, RA)],
                                sh.at[pl.ds(start, RA)])
            else:
                pltpu.sync_copy(sh.at[pl.ds(start, RA)],
                                out_hbm.at[cid, pl.ds(start, RA)])

        @pl.when(sid == NS - 1)
        def _():
            if phase == 0:
                pltpu.sync_copy(zeros_hbm,
                                sh.at[pl.ds((NS - 1) * RA, RB)])
            else:
                pltpu.sync_copy(sh.at[pl.ds((NS - 1) * RA, RB)],
                                out_hbm.at[cid, pl.ds((NS - 1) * RA, RB)])

    @functools.partial(
        pl.kernel,
        out_type=jax.ShapeDtypeStruct((NC, N, 16), jnp.float32),
        mesh=mesh,
        scratch_types=[
            [pltpu.VMEM((CD,), jnp.int32)] * KD,
            pltpu.VMEM((CD, 16), jnp.float32),
            pltpu.VMEM_SHARED((N, 16), jnp.float32),
            [pltpu.SemaphoreType.DMA] * KD,
            [pltpu.SemaphoreType.DMA] * KD,
        ],
        compiler_params=pltpu.CompilerParams(use_tc_tiling_on_sc=False),
    )
    def sc_degree(col_hbm, ones_hbm, zeros_hbm, out_hbm, cidx_vs, ones_v,
                  deg_sh, isems, ssems):
        cid = lax.axis_index("c")
        sid = lax.axis_index("s")
        wid = sid * NC + cid
        _init_and_readback(sid, cid, zeros_hbm, deg_sh, out_hbm, 0)
        pltpu.sync_copy(ones_hbm, ones_v)

        def isrc(i):
            return col_hbm.at[pl.ds(pl.multiple_of(wid * EPW + i * CD, 8), CD)]

        def iload(i, b):
            pltpu.async_copy(isrc(i), cidx_vs[b], isems[b])

        def iwait(i, b):
            pltpu.make_async_copy(isrc(i), cidx_vs[b], isems[b]).wait()

        def scatter(b):
            pltpu.async_copy(ones_v, deg_sh.at[cidx_vs[b]], ssems[b], add=True)

        def scatter_wait(b):
            pltpu.make_async_copy(ones_v, deg_sh.at[cidx_vs[b]],
                                  ssems[b]).wait()

        for b in range(KD):
            iload(b, b)
        plsc.subcore_barrier()

        def group(j, carry):
            for b in range(KD):
                i = j * KD + b
                iwait(i, b)
                scatter(b)

            @pl.when(j < NGD - 1)
            def _():
                for b in range(KD):
                    scatter_wait(b)
                    iload((j + 1) * KD + b, b)

            return carry

        lax.fori_loop(0, NGD, group, 0)
        for b in range(KD):
            scatter_wait(b)
        plsc.subcore_barrier()
        _init_and_readback(sid, cid, zeros_hbm, deg_sh, out_hbm, 1)

    @functools.partial(
        pl.kernel,
        out_type=jax.ShapeDtypeStruct((NC, N, DH), jnp.float32),
        mesh=mesh,
        scratch_types=[
            [pltpu.VMEM((K * CH,), jnp.int32)] * 2,
            [pltpu.VMEM((CH,), jnp.int32)] * (2 * K),
            pltpu.VMEM((K * CH, DH), jnp.float32),
            pltpu.VMEM_SHARED((N, DH), jnp.float32),
            [pltpu.SemaphoreType.DMA] * K,
            [pltpu.SemaphoreType.DMA] * K,
            [pltpu.SemaphoreType.DMA] * 2,
            [pltpu.SemaphoreType.DMA] * 2,
        ],
        compiler_params=pltpu.CompilerParams(use_tc_tiling_on_sc=False),
    )
    def sc_spmm(g3_hbm, row_hbm, col_hbm, zeros_hbm, out_hbm,
                ridx_vs, cidx_vs, rows_v, agg_sh, gsems, ssems, irsems, icsems):
        cid = lax.axis_index("c")
        sid = lax.axis_index("s")
        _init_and_readback(sid, cid, zeros_hbm, agg_sh, out_hbm, 0)

        SLAB = K * CH

        def slab_refs(j, d):
            base = pl.multiple_of(sid * EPT + j * SLAB, 8)
            refs = [(row_hbm.at[pl.ds(base, SLAB)], ridx_vs[d], irsems[d])]
            for b in range(K):
                cb = pl.multiple_of(base + b * CH, 8)
                refs.append((col_hbm.at[pl.ds(cb, CH)], cidx_vs[d * K + b],
                             icsems[d]))
            return refs

        def slab_load(j, d):
            for s, dst, sem in slab_refs(j, d):
                pltpu.async_copy(s, dst, sem)

        def slab_wait(j, d):
            for s, dst, sem in slab_refs(j, d):
                pltpu.make_async_copy(s, dst, sem).wait()

        def gsrc(d, b):
            return g3_hbm.at[cid].at[ridx_vs[d].at[pl.ds(b * CH, CH)]]

        def rbuf(b):
            return rows_v.at[pl.ds(b * CH, CH)]

        def gather(d, b):
            pltpu.async_copy(gsrc(d, b), rbuf(b), gsems[b])

        def gather_wait(d, b):
            pltpu.make_async_copy(gsrc(d, b), rbuf(b), gsems[b]).wait()

        def scatter(d, b):
            pltpu.async_copy(rbuf(b), agg_sh.at[cidx_vs[d * K + b]],
                             ssems[b], add=True)

        def scatter_wait(d, b):
            pltpu.make_async_copy(rbuf(b), agg_sh.at[cidx_vs[d * K + b]],
                                  ssems[b]).wait()

        slab_load(0, 0)
        slab_wait(0, 0)
        plsc.subcore_barrier()
        for b in range(K):
            gather(0, b)

        def one_group(j, d):
            nd = 1 - d

            @pl.when(j < NG - 1)
            def _():
                slab_load(j + 1, nd)

            for b in range(K):
                gather_wait(d, b)
                scatter(d, b)

            @pl.when(j < NG - 1)
            def _():
                slab_wait(j + 1, nd)
                for b in range(K):
                    scatter_wait(d, b)
                    gather(nd, b)

        def pair(t, carry):
            one_group(2 * t, 0)
            one_group(2 * t + 1, 1)
            return carry

        lax.fori_loop(0, NG // 2, pair, 0)
        for b in range(K):
            scatter_wait((NG - 1) % 2, b)
        plsc.subcore_barrier()
        _init_and_readback(sid, cid, zeros_hbm, agg_sh, out_hbm, 1)

    return sc_degree, sc_spmm


R = 2000
G = N // R


def _dot_t(a, w):
    return lax.dot_general(a, w, (((1,), (1,)), ((), ())),
                           preferred_element_type=jnp.float32)


def _ln(h, g, b):
    mu = jnp.mean(h, axis=-1, keepdims=True)
    var = jnp.mean((h - mu) * (h - mu), axis=-1, keepdims=True)
    return (h - mu) * lax.rsqrt(var + 1e-5) * g + b


def _ka_body(x_ref, deg_ref, tfw, tfb, ln0g, ln0b, wq, wqb, wk, wkb, wv, wvb,
             gfw, gfb, bn0g, bn0b,
             layer0_ref, qs_ref, g0_ref, g1s_ref, kvs_ref, vsum_ref, ksum_ref):
    i = pl.program_id(0)
    xb = x_ref[...]
    h = _dot_t(xb, tfw[...]) + tfb[...]
    h = jnp.maximum(_ln(h, ln0g[...], ln0b[...]), 0.0)
    layer0_ref[...] = h
    qs = _dot_t(h, wq[...]) + wqb[...]
    kk = _dot_t(h, wk[...]) + wkb[...]
    vs = _dot_t(h, wv[...]) + wvb[...]
    qs = qs * lax.rsqrt(jnp.sum(qs * qs, axis=-1, keepdims=True))
    kk = kk * lax.rsqrt(jnp.sum(kk * kk, axis=-1, keepdims=True))
    qs_ref[...] = qs
    kv = lax.dot_general(kk, vs, (((0,), (0,)), ((), ())),
                         preferred_element_type=jnp.float32)
    vsum = jnp.sum(vs, axis=0, keepdims=True)
    ksum = jnp.sum(kk, axis=0, keepdims=True)

    @pl.when(i == 0)
    def _():
        kvs_ref[...] = kv
        vsum_ref[...] = vsum
        ksum_ref[...] = ksum

    @pl.when(i > 0)
    def _():
        kvs_ref[...] += kv
        vsum_ref[...] += vsum
        ksum_ref[...] += ksum

    g0 = _dot_t(xb, gfw[...]) + gfb[...]
    g0 = jnp.maximum(g0 * _BN_INV * bn0g[...] + bn0b[...], 0.0)
    g0_ref[...] = g0
    _split_store(g1s_ref, _scale_from_deg(deg_ref) * g0)


def _scale_from_deg(deg_ref):
    degb = deg_ref[0, :, 0:1] + deg_ref[1, :, 0:1]
    return jnp.where(degb > 0.0, lax.rsqrt(degb), 0.0)


def _split_store(out_ref, t):
    out_ref[0] = t[:, :DH]
    out_ref[1] = t[:, DH:]


def _kb_body(p_ref, deg_ref, g0_ref, w, b, bng, bnb, out_ref):
    s = _scale_from_deg(deg_ref)
    agg = jnp.concatenate([p_ref[0], p_ref[1]], axis=-1) * s
    t = _dot_t(agg, w[...]) + b[...]
    t = jnp.maximum(t * _BN_INV * bng[...] + bnb[...], 0.0) + g0_ref[...]
    _split_store(out_ref, s * t)


def _kd_body(p_ref, deg_ref, g0_ref, qs_ref, layer0_ref,
             kvs_ref, vsum_ref, ksum_ref,
             w, b, bng, bnb, ln1g, ln1b, ow, ob, out_ref):
    s = _scale_from_deg(deg_ref)
    agg = jnp.concatenate([p_ref[0], p_ref[1]], axis=-1) * s
    t = _dot_t(agg, w[...]) + b[...]
    x2 = jnp.maximum(t * _BN_INV * bng[...] + bnb[...], 0.0) + g0_ref[...]

    qs = qs_ref[...]
    num = jnp.dot(qs, kvs_ref[...], preferred_element_type=jnp.float32)
    num = num + jnp.float32(N) * vsum_ref[...]
    den = jnp.sum(qs * ksum_ref[...], axis=-1, keepdims=True)
    den = den + jnp.float32(2 * N)
    h = (num / den + layer0_ref[...]) * 0.5
    x1 = jnp.maximum(_ln(h, ln1g[...], ln1b[...]), 0.0)

    out_ref[...] = _dot_t(0.8 * x2 + 0.2 * x1, ow[...]) + ob[...]


_WSPEC = pl.BlockSpec((D, D), lambda i: (0, 0))
_BSPEC = pl.BlockSpec((1, D), lambda i: (0, 0))
_XSPEC = pl.BlockSpec((R, D), lambda i: (i, 0))
_DEGSPEC = pl.BlockSpec((NC, R, 16), lambda i: (0, i, 0))
_PSPEC = pl.BlockSpec((NC, R, DH), lambda i: (0, i, 0))

_f32 = jnp.float32


def _ka(x, deg, tfw, tfb, ln0g, ln0b, wq, wqb, wk, wkb, wv, wvb,
        gfw, gfb, bn0g, bn0b):
    outs = (
        jax.ShapeDtypeStruct((N, D), _f32),
        jax.ShapeDtypeStruct((N, D), _f32),
        jax.ShapeDtypeStruct((N, D), _f32),
        jax.ShapeDtypeStruct((NC, N, DH), _f32),
        jax.ShapeDtypeStruct((D, D), _f32),
        jax.ShapeDtypeStruct((1, D), _f32),
        jax.ShapeDtypeStruct((1, D), _f32),
    )
    in_specs = [_XSPEC, _DEGSPEC, _WSPEC, _BSPEC, _BSPEC, _BSPEC,
                _WSPEC, _BSPEC, _WSPEC, _BSPEC, _WSPEC, _BSPEC,
                _WSPEC, _BSPEC, _BSPEC, _BSPEC]
    out_specs = (_XSPEC, _XSPEC, _XSPEC, _PSPEC, _WSPEC, _BSPEC, _BSPEC)
    return pl.pallas_call(
        _ka_body, grid=(G,), in_specs=in_specs, out_specs=out_specs,
        out_shape=outs)(x, deg, tfw, tfb, ln0g, ln0b, wq, wqb, wk, wkb,
                        wv, wvb, gfw, gfb, bn0g, bn0b)


def _kb(p, deg, g0, w, b, bng, bnb):
    return pl.pallas_call(
        _kb_body, grid=(G,),
        in_specs=[_PSPEC, _DEGSPEC, _XSPEC, _WSPEC, _BSPEC, _BSPEC, _BSPEC],
        out_specs=_PSPEC,
        out_shape=jax.ShapeDtypeStruct((NC, N, DH), _f32))(p, deg, g0, w, b,
                                                           bng, bnb)


def _kd(p, deg, g0, qs, layer0, kvs, vsum, ksum, w, b, bng, bnb,
        ln1g, ln1b, ow, ob):
    return pl.pallas_call(
        _kd_body, grid=(G,),
        in_specs=[_PSPEC, _DEGSPEC, _XSPEC, _XSPEC, _XSPEC,
                  _WSPEC, _BSPEC, _BSPEC,
                  _WSPEC, _BSPEC, _BSPEC, _BSPEC, _BSPEC, _BSPEC,
                  _WSPEC, _BSPEC],
        out_specs=_XSPEC,
        out_shape=jax.ShapeDtypeStruct((N, D), _f32))(
            p, deg, g0, qs, layer0, kvs, vsum, ksum,
            w, b, bng, bnb, ln1g, ln1b, ow, ob)


def kernel(x, edge_index, t_fc_w, t_fc_b, t_ln0_g, t_ln0_b, t_wq_w, t_wq_b,
           t_wk_w, t_wk_b, t_wv_w, t_wv_b, t_ln1_g, t_ln1_b, g_fc_w, g_fc_b,
           g_bn0_g, g_bn0_b, g_w1_w, g_w1_b, g_bn1_g, g_bn1_b, g_w2_w, g_w2_b,
           g_bn2_g, g_bn2_b, out_w, out_b):
    row = edge_index[0]
    col = edge_index[1]

    r2 = lambda v: v.reshape(1, D)

    ones16 = jnp.ones((CD, 16), _f32)
    zerosD = jnp.zeros((RB, DH), _f32)
    zeros16 = jnp.zeros((RB, 16), _f32)

    sc_degree, sc_spmm = _sc_kernels()
    deg = sc_degree(col, ones16, zeros16)

    layer0, qs, g0, g1s, kvs, vsum, ksum = _ka(
        x, deg, t_fc_w, r2(t_fc_b), r2(t_ln0_g), r2(t_ln0_b),
        t_wq_w, r2(t_wq_b), t_wk_w, r2(t_wk_b), t_wv_w, r2(t_wv_b),
        g_fc_w, r2(g_fc_b), r2(g_bn0_g), r2(g_bn0_b))

    p1 = sc_spmm(g1s, row, col, zerosD)
    g2s = _kb(p1, deg, g0, g_w1_w, r2(g_w1_b), r2(g_bn1_g), r2(g_bn1_b))
    p2 = sc_spmm(g2s, row, col, zerosD)
    out = _kd(p2, deg, g0, qs, layer0, kvs, vsum, ksum,
              g_w2_w, r2(g_w2_b), r2(g_bn2_g), r2(g_bn2_b),
              r2(t_ln1_g), r2(t_ln1_b), out_w, r2(out_b))
    return out


# speedup vs baseline: 1.0729x; 1.0682x over previous
import functools

import jax
import jax.numpy as jnp
from jax import lax
from jax.experimental import pallas as pl
from jax.experimental.pallas import tpu as pltpu
from jax.experimental.pallas import tpu_sc as plsc

N = 10000
E = 320000
D = 128

NC = 2
NS = 16
NW = NC * NS
EPW = E // NW
CH = 40
NCHUNK = EPW // CH
K = 5
NG = NCHUNK // K
CD = 80
ND = EPW // CD
KD = 5
NGD = ND // KD
RA = 624
RB = N - (NS - 1) * RA

_BN_INV = 0.9999950000374997


@functools.lru_cache(maxsize=None)
def _sc_kernels():
    mesh = plsc.VectorSubcoreMesh(core_axis_name="c", subcore_axis_name="s")

    def _init_and_readback(sid, cid, zeros_hbm, sh, out_hbm, phase):
        start = pl.multiple_of(sid * RA, 8)

        @pl.when(sid < NS - 1)
        def _():
            if phase == 0:
                pltpu.sync_copy(zeros_hbm.at[pl.ds(0, RA)],
                                sh.at[pl.ds(start, RA)])
            else:
                pltpu.sync_copy(sh.at[pl.ds(start, RA)],
                                out_hbm.at[cid, pl.ds(start, RA)])

        @pl.when(sid == NS - 1)
        def _():
            if phase == 0:
                pltpu.sync_copy(zeros_hbm,
                                sh.at[pl.ds((NS - 1) * RA, RB)])
            else:
                pltpu.sync_copy(sh.at[pl.ds((NS - 1) * RA, RB)],
                                out_hbm.at[cid, pl.ds((NS - 1) * RA, RB)])

    @functools.partial(
        pl.kernel,
        out_type=jax.ShapeDtypeStruct((NC, N, 16), jnp.float32),
        mesh=mesh,
        scratch_types=[
            [pltpu.VMEM((CD,), jnp.int32)] * KD,
            pltpu.VMEM((CD, 16), jnp.float32),
            pltpu.VMEM_SHARED((N, 16), jnp.float32),
            [pltpu.SemaphoreType.DMA] * KD,
            [pltpu.SemaphoreType.DMA] * KD,
        ],
        compiler_params=pltpu.CompilerParams(use_tc_tiling_on_sc=False),
    )
    def sc_degree(col_hbm, ones_hbm, zeros_hbm, out_hbm, cidx_vs, ones_v,
                  deg_sh, isems, ssems):
        cid = lax.axis_index("c")
        sid = lax.axis_index("s")
        wid = sid * NC + cid
        _init_and_readback(sid, cid, zeros_hbm, deg_sh, out_hbm, 0)
        pltpu.sync_copy(ones_hbm, ones_v)

        def isrc(i):
            return col_hbm.at[pl.ds(pl.multiple_of(wid * EPW + i * CD, 8), CD)]

        def iload(i, b):
            pltpu.async_copy(isrc(i), cidx_vs[b], isems[b])

        def iwait(i, b):
            pltpu.make_async_copy(isrc(i), cidx_vs[b], isems[b]).wait()

        def scatter(b):
            pltpu.async_copy(ones_v, deg_sh.at[cidx_vs[b]], ssems[b], add=True)

        def scatter_wait(b):
            pltpu.make_async_copy(ones_v, deg_sh.at[cidx_vs[b]],
                                  ssems[b]).wait()

        for b in range(KD):
            iload(b, b)
        plsc.subcore_barrier()

        def group(j, carry):
            for b in range(KD):
                i = j * KD + b
                iwait(i, b)
                scatter(b)

            @pl.when(j < NGD - 1)
            def _():
                for b in range(KD):
                    scatter_wait(b)
                    iload((j + 1) * KD + b, b)

            return carry

        lax.fori_loop(0, NGD, group, 0)
        for b in range(KD):
            scatter_wait(b)
        plsc.subcore_barrier()
        _init_and_readback(sid, cid, zeros_hbm, deg_sh, out_hbm, 1)

    @functools.partial(
        pl.kernel,
        out_type=jax.ShapeDtypeStruct((NC, N, D), jnp.float32),
        mesh=mesh,
        scratch_types=[
            [pltpu.VMEM((K * CH,), jnp.int32)] * 2,
            [pltpu.VMEM((CH,), jnp.int32)] * (2 * K),
            pltpu.VMEM((K * CH, D), jnp.float32),
            pltpu.VMEM_SHARED((N, D), jnp.float32),
            [pltpu.SemaphoreType.DMA] * K,
            [pltpu.SemaphoreType.DMA] * K,
            [pltpu.SemaphoreType.DMA] * 2,
            [pltpu.SemaphoreType.DMA] * 2,
        ],
    )
    def sc_spmm(g_hbm, row_hbm, col_hbm, zeros_hbm, out_hbm,
                ridx_vs, cidx_vs, rows_v, agg_sh, gsems, ssems, irsems, icsems):
        cid = lax.axis_index("c")
        sid = lax.axis_index("s")
        wid = sid * NC + cid
        _init_and_readback(sid, cid, zeros_hbm, agg_sh, out_hbm, 0)

        SLAB = K * CH

        def slab_refs(j, d):
            base = pl.multiple_of(wid * EPW + j * SLAB, 8)
            refs = [(row_hbm.at[pl.ds(base, SLAB)], ridx_vs[d], irsems[d])]
            for b in range(K):
                cb = pl.multiple_of(base + b * CH, 8)
                refs.append((col_hbm.at[pl.ds(cb, CH)], cidx_vs[d * K + b],
                             icsems[d]))
            return refs

        def slab_load(j, d):
            for s, dst, sem in slab_refs(j, d):
                pltpu.async_copy(s, dst, sem)

        def slab_wait(j, d):
            for s, dst, sem in slab_refs(j, d):
                pltpu.make_async_copy(s, dst, sem).wait()

        def gsrc(d, b):
            return g_hbm.at[ridx_vs[d].at[pl.ds(b * CH, CH)]]

        def rbuf(b):
            return rows_v.at[pl.ds(b * CH, CH)]

        def gather(d, b):
            pltpu.async_copy(gsrc(d, b), rbuf(b), gsems[b])

        def gather_wait(d, b):
            pltpu.make_async_copy(gsrc(d, b), rbuf(b), gsems[b]).wait()

        def scatter(d, b):
            pltpu.async_copy(rbuf(b), agg_sh.at[cidx_vs[d * K + b]],
                             ssems[b], add=True)

        def scatter_wait(d, b):
            pltpu.make_async_copy(rbuf(b), agg_sh.at[cidx_vs[d * K + b]],
                                  ssems[b]).wait()

        slab_load(0, 0)
        slab_wait(0, 0)
        plsc.subcore_barrier()
        for b in range(K):
            gather(0, b)

        def one_group(j, d):
            nd = 1 - d

            @pl.when(j < NG - 1)
            def _():
                slab_load(j + 1, nd)

            for b in range(K):
                gather_wait(d, b)
                scatter(d, b)

            @pl.when(j < NG - 1)
            def _():
                slab_wait(j + 1, nd)
                for b in range(K):
                    scatter_wait(d, b)
                    gather(nd, b)

        def pair(t, carry):
            one_group(2 * t, 0)
            one_group(2 * t + 1, 1)
            return carry

        lax.fori_loop(0, NG // 2, pair, 0)
        for b in range(K):
            scatter_wait((NG - 1) % 2, b)
        plsc.subcore_barrier()
        _init_and_readback(sid, cid, zeros_hbm, agg_sh, out_hbm, 1)

    return sc_degree, sc_spmm


R = 2000
G = N // R


def _dot_t(a, w):
    return lax.dot_general(a, w, (((1,), (1,)), ((), ())),
                           preferred_element_type=jnp.float32)


def _ln(h, g, b):
    mu = jnp.mean(h, axis=-1, keepdims=True)
    var = jnp.mean((h - mu) * (h - mu), axis=-1, keepdims=True)
    return (h - mu) * lax.rsqrt(var + 1e-5) * g + b


def _ka_body(x_ref, tfw, tfb, ln0g, ln0b, wq, wqb, wk, wkb, wv, wvb,
             gfw, gfb, bn0g, bn0b,
             layer0_ref, qs_ref, g0_ref, kvs_ref, vsum_ref, ksum_ref):
    i = pl.program_id(0)
    xb = x_ref[...]
    h = _dot_t(xb, tfw[...]) + tfb[...]
    h = jnp.maximum(_ln(h, ln0g[...], ln0b[...]), 0.0)
    layer0_ref[...] = h
    qs = _dot_t(h, wq[...]) + wqb[...]
    kk = _dot_t(h, wk[...]) + wkb[...]
    vs = _dot_t(h, wv[...]) + wvb[...]
    qs = qs * lax.rsqrt(jnp.sum(qs * qs, axis=-1, keepdims=True))
    kk = kk * lax.rsqrt(jnp.sum(kk * kk, axis=-1, keepdims=True))
    qs_ref[...] = qs
    kv = lax.dot_general(kk, vs, (((0,), (0,)), ((), ())),
                         preferred_element_type=jnp.float32)
    vsum = jnp.sum(vs, axis=0, keepdims=True)
    ksum = jnp.sum(kk, axis=0, keepdims=True)

    @pl.when(i == 0)
    def _():
        kvs_ref[...] = kv
        vsum_ref[...] = vsum
        ksum_ref[...] = ksum

    @pl.when(i > 0)
    def _():
        kvs_ref[...] += kv
        vsum_ref[...] += vsum
        ksum_ref[...] += ksum

    g0 = _dot_t(xb, gfw[...]) + gfb[...]
    g0 = jnp.maximum(g0 * _BN_INV * bn0g[...] + bn0b[...], 0.0)
    g0_ref[...] = g0


def _scale_from_deg(deg_ref):
    degb = deg_ref[0, :, 0:1] + deg_ref[1, :, 0:1]
    return jnp.where(degb > 0.0, lax.rsqrt(degb), 0.0)


def _ks_body(deg_ref, g0_ref, out_ref):
    out_ref[...] = _scale_from_deg(deg_ref) * g0_ref[...]


def _kb_body(p_ref, deg_ref, g0_ref, w, b, bng, bnb, out_ref):
    s = _scale_from_deg(deg_ref)
    agg = (p_ref[0] + p_ref[1]) * s
    t = _dot_t(agg, w[...]) + b[...]
    t = jnp.maximum(t * _BN_INV * bng[...] + bnb[...], 0.0) + g0_ref[...]
    out_ref[...] = s * t


def _kx_body(qs_ref, layer0_ref, kvs_ref, vsum_ref, ksum_ref,
             ln1g, ln1b, x1_ref):
    qs = qs_ref[...]
    num = jnp.dot(qs, kvs_ref[...], preferred_element_type=jnp.float32)
    num = num + jnp.float32(N) * vsum_ref[...]
    den = jnp.sum(qs * ksum_ref[...], axis=-1, keepdims=True)
    den = den + jnp.float32(2 * N)
    h = (num / den + layer0_ref[...]) * 0.5
    x1_ref[...] = jnp.maximum(_ln(h, ln1g[...], ln1b[...]), 0.0)


def _kd_body(p_ref, deg_ref, g0_ref, x1_ref,
             w, b, bng, bnb, ow, ob, out_ref):
    s = _scale_from_deg(deg_ref)
    agg = (p_ref[0] + p_ref[1]) * s
    t = _dot_t(agg, w[...]) + b[...]
    x2 = jnp.maximum(t * _BN_INV * bng[...] + bnb[...], 0.0) + g0_ref[...]
    out_ref[...] = _dot_t(0.8 * x2 + 0.2 * x1_ref[...], ow[...]) + ob[...]


_WSPEC = pl.BlockSpec((D, D), lambda i: (0, 0))
_BSPEC = pl.BlockSpec((1, D), lambda i: (0, 0))
_XSPEC = pl.BlockSpec((R, D), lambda i: (i, 0))
_DEGSPEC = pl.BlockSpec((NC, R, 16), lambda i: (0, i, 0))
_PSPEC = pl.BlockSpec((NC, R, D), lambda i: (0, i, 0))

_f32 = jnp.float32


def _ka(x, tfw, tfb, ln0g, ln0b, wq, wqb, wk, wkb, wv, wvb, gfw, gfb, bn0g, bn0b):
    outs = (
        jax.ShapeDtypeStruct((N, D), _f32),
        jax.ShapeDtypeStruct((N, D), _f32),
        jax.ShapeDtypeStruct((N, D), _f32),
        jax.ShapeDtypeStruct((D, D), _f32),
        jax.ShapeDtypeStruct((1, D), _f32),
        jax.ShapeDtypeStruct((1, D), _f32),
    )
    in_specs = [_XSPEC, _WSPEC, _BSPEC, _BSPEC, _BSPEC,
                _WSPEC, _BSPEC, _WSPEC, _BSPEC, _WSPEC, _BSPEC,
                _WSPEC, _BSPEC, _BSPEC, _BSPEC]
    out_specs = (_XSPEC, _XSPEC, _XSPEC, _WSPEC, _BSPEC, _BSPEC)
    return pl.pallas_call(
        _ka_body, grid=(G,), in_specs=in_specs, out_specs=out_specs,
        out_shape=outs)(x, tfw, tfb, ln0g, ln0b, wq, wqb, wk, wkb, wv, wvb,
                        gfw, gfb, bn0g, bn0b)


def _ks(deg, g0):
    return pl.pallas_call(
        _ks_body, grid=(G,), in_specs=[_DEGSPEC, _XSPEC], out_specs=_XSPEC,
        out_shape=jax.ShapeDtypeStruct((N, D), _f32))(deg, g0)


def _kb(p, deg, g0, w, b, bng, bnb):
    return pl.pallas_call(
        _kb_body, grid=(G,),
        in_specs=[_PSPEC, _DEGSPEC, _XSPEC, _WSPEC, _BSPEC, _BSPEC, _BSPEC],
        out_specs=_XSPEC,
        out_shape=jax.ShapeDtypeStruct((N, D), _f32))(p, deg, g0, w, b, bng, bnb)


def _kx(qs, layer0, kvs, vsum, ksum, ln1g, ln1b):
    return pl.pallas_call(
        _kx_body, grid=(G,),
        in_specs=[_XSPEC, _XSPEC, _WSPEC, _BSPEC, _BSPEC, _BSPEC, _BSPEC],
        out_specs=_XSPEC,
        out_shape=jax.ShapeDtypeStruct((N, D), _f32))(
            qs, layer0, kvs, vsum, ksum, ln1g, ln1b)


def _kd(p, deg, g0, x1, w, b, bng, bnb, ow, ob):
    return pl.pallas_call(
        _kd_body, grid=(G,),
        in_specs=[_PSPEC, _DEGSPEC, _XSPEC, _XSPEC,
                  _WSPEC, _BSPEC, _BSPEC, _BSPEC, _WSPEC, _BSPEC],
        out_specs=_XSPEC,
        out_shape=jax.ShapeDtypeStruct((N, D), _f32))(
            p, deg, g0, x1, w, b, bng, bnb, ow, ob)


def kernel(x, edge_index, t_fc_w, t_fc_b, t_ln0_g, t_ln0_b, t_wq_w, t_wq_b,
           t_wk_w, t_wk_b, t_wv_w, t_wv_b, t_ln1_g, t_ln1_b, g_fc_w, g_fc_b,
           g_bn0_g, g_bn0_b, g_w1_w, g_w1_b, g_bn1_g, g_bn1_b, g_w2_w, g_w2_b,
           g_bn2_g, g_bn2_b, out_w, out_b):
    row = edge_index[0]
    col = edge_index[1]

    r2 = lambda v: v.reshape(1, D)

    ones16 = jnp.ones((CD, 16), _f32)
    zerosD = jnp.zeros((RB, D), _f32)
    zeros16 = jnp.zeros((RB, 16), _f32)

    layer0, qs, g0, kvs, vsum, ksum = _ka(
        x, t_fc_w, r2(t_fc_b), r2(t_ln0_g), r2(t_ln0_b),
        t_wq_w, r2(t_wq_b), t_wk_w, r2(t_wk_b), t_wv_w, r2(t_wv_b),
        g_fc_w, r2(g_fc_b), r2(g_bn0_g), r2(g_bn0_b))

    sc_degree, sc_spmm = _sc_kernels()
    deg = sc_degree(col, ones16, zeros16)

    g1s = _ks(deg, g0)
    p1 = sc_spmm(g1s, row, col, zerosD)
    x1 = _kx(qs, layer0, kvs, vsum, ksum, r2(t_ln1_g), r2(t_ln1_b))
    g2s = _kb(p1, deg, g0, g_w1_w, r2(g_w1_b), r2(g_bn1_g), r2(g_bn1_b))
    p2 = sc_spmm(g2s, row, col, zerosD)
    out = _kd(p2, deg, g0, x1,
              g_w2_w, r2(g_w2_b), r2(g_bn2_g), r2(g_bn2_b),
              out_w, r2(out_b))
    return out
